# baseline (speedup 1.0000x reference)
import functools

import jax
import jax.numpy as jnp
from jax import lax
from jax.experimental import pallas as pl
from jax.experimental.pallas import tpu as pltpu
from jax.experimental.pallas import tpu_sc as plsc

F = 32
NB = 16
NIT = 2
NRES = 3
CUTOFF = 6.0

NC = 2
NS = 16
NW = NC * NS
C = 128

_SC_MESH = dict(core_axis_name="c", subcore_axis_name="s",
                num_cores=NC, num_subcores=NS)
_SC_PARAMS = pltpu.CompilerParams(use_tc_tiling_on_sc=False,
                                  needs_layout_passes=False)


def _vgather(x, idx):
    return lax.gather(
        x, idx[:, None],
        lax.GatherDimensionNumbers(offset_dims=(), collapsed_slice_dims=(0,),
                                   start_index_map=(0,)),
        (1,), mode=lax.GatherScatterMode.PROMISE_IN_BOUNDS)


def _geom_body(n_chunks, pos_hbm, srcr_hbm, dstr_hbm, rsq_hbm,
               src_v, dst_v, ps_v, pd_v, rs_v, sem1, sem2):
    cid = lax.axis_index("c")
    sid = lax.axis_index("s")
    wid = cid * NS + sid
    pltpu.sync_copy(srcr_hbm.at[wid], src_v)
    pltpu.sync_copy(dstr_hbm.at[wid], dst_v)
    lanes = lax.iota(jnp.int32, 16)
    ix = jnp.zeros((16,), jnp.int32)
    iy = jnp.full((16,), 1, jnp.int32)
    iz = jnp.full((16,), 2, jnp.int32)

    @pl.loop(0, n_chunks)
    def _chunk(j):
        cp1 = pltpu.async_copy(pos_hbm.at[src_v.at[j]], ps_v, sem1)
        cp2 = pltpu.async_copy(pos_hbm.at[dst_v.at[j]], pd_v, sem2)
        cp1.wait()
        cp2.wait()

        @pl.loop(0, C // 16)
        def _grp(gq):
            acc = jnp.zeros((16,), jnp.float32)
            for sub in range(16):
                r = gq * 16 + sub
                dv = ps_v[r, :] - pd_v[r, :]
                d2 = dv * dv
                rs = _vgather(d2, ix) + _vgather(d2, iy) + _vgather(d2, iz)
                acc = jnp.where(lanes == sub, rs, acc)
            rs_v[pl.ds(gq * 16, 16)] = acc

        pltpu.sync_copy(rs_v, rsq_hbm.at[wid * n_chunks + j])


def _make_geom(n_pad, n_chunks):
    return functools.partial(
        pl.kernel,
        out_type=jax.ShapeDtypeStruct((NW * n_chunks, C), jnp.float32),
        mesh=plsc.VectorSubcoreMesh(**_SC_MESH),
        compiler_params=_SC_PARAMS,
        scratch_types=[
            pltpu.VMEM((n_chunks, C), jnp.int32),
            pltpu.VMEM((n_chunks, C), jnp.int32),
            pltpu.VMEM((C, 16), jnp.float32),
            pltpu.VMEM((C, 16), jnp.float32),
            pltpu.VMEM((C,), jnp.float32),
            pltpu.SemaphoreType.DMA,
            pltpu.SemaphoreType.DMA,
        ],
    )(functools.partial(_geom_body, n_chunks))


def _msg_body(n_real, n_chunks, rpt, zr, ob_n, ob_c,
              x_hbm, g_hbm, srcr_hbm, dstr_hbm, agg_hbm,
              acc, si_s, di_s, xv0, xv1, gv0, gv1, mv0, mv1,
              sx0, sx1, sg0, sg1, ss0, ss1):
    cid = lax.axis_index("c")
    sid = lax.axis_index("s")
    wid = cid * NS + sid
    zero16 = jnp.zeros((16,), jnp.float32)

    @pl.loop(0, C)
    def _zrow(r):
        mv0[r, pl.ds(0, 16)] = zero16
        mv0[r, pl.ds(16, 16)] = zero16

    @pl.loop(0, rpt // zr)
    def _zacc(k):
        pltpu.sync_copy(mv0.at[pl.ds(0, zr)],
                        acc.at[pl.ds(sid * rpt + k * zr, zr)])

    plsc.subcore_barrier()

    def mult(xv, gv, mv):
        @pl.loop(0, C)
        def _mul(r):
            lo, hi = plsc.unpack(gv[r, :], format=plsc.PackFormat.INTERLEAVED)
            mv[r, pl.ds(0, 16)] = xv[r, pl.ds(0, 16)] * lo
            mv[r, pl.ds(16, 16)] = xv[r, pl.ds(16, 16)] * hi

    def drain_scatter(sem, mv):
        pltpu.make_async_copy(x_hbm.at[pl.ds(0, C)], mv, sem).wait()

    @pl.loop(0, ob_n)
    def _ob(ob):
        base = wid * n_chunks + ob * ob_c

        @pl.when(ob > 0)
        def _drain_prev():
            drain_scatter(ss0, mv0)
            drain_scatter(ss1, mv1)

        pltpu.sync_copy(srcr_hbm.at[(wid * ob_n + ob)], si_s)
        pltpu.sync_copy(dstr_hbm.at[(wid * ob_n + ob)], di_s)

        def fire(j, xv, gv, sx, sg):
            pltpu.async_copy(x_hbm.at[si_s.at[j]], xv, sx)
            pltpu.async_copy(g_hbm.at[base + j], gv, sg)

        def wait_loads(xv, gv, sx, sg):
            pltpu.make_async_copy(x_hbm.at[pl.ds(0, C)], xv, sx).wait()
            pltpu.make_async_copy(g_hbm.at[0], gv, sg).wait()

        fire(0, xv0, gv0, sx0, sg0)
        fire(1, xv1, gv1, sx1, sg1)

        @pl.loop(0, ob_c // 2)
        def _pair(p):
            j0 = 2 * p
            wait_loads(xv0, gv0, sx0, sg0)
            mult(xv0, gv0, mv0)

            @pl.when(p < ob_c // 2 - 1)
            def _pf0():
                fire(j0 + 2, xv0, gv0, sx0, sg0)

            @pl.when(p > 0)
            def _dr0():
                drain_scatter(ss0, mv0)

            pltpu.async_copy(mv0, acc.at[di_s.at[j0]], ss0, add=True)

            wait_loads(xv1, gv1, sx1, sg1)
            mult(xv1, gv1, mv1)

            @pl.when(p < ob_c // 2 - 1)
            def _pf1():
                fire(j0 + 3, xv1, gv1, sx1, sg1)

            @pl.when(p > 0)
            def _dr1():
                drain_scatter(ss1, mv1)

            pltpu.async_copy(mv1, acc.at[di_s.at[j0 + 1]], ss1, add=True)

    drain_scatter(ss0, mv0)
    drain_scatter(ss1, mv1)
    plsc.subcore_barrier()
    pltpu.sync_copy(acc.at[pl.ds(sid * rpt, rpt)],
                    agg_hbm.at[pl.ds(cid * n_real + sid * rpt, rpt)])


def _make_msg(n_real, n_chunks):
    rpt = n_real // NS
    zr = 125
    ob_c = 14
    ob_n = n_chunks // ob_c
    return functools.partial(
        pl.kernel,
        out_type=jax.ShapeDtypeStruct((NC * n_real, F), jnp.float32),
        mesh=plsc.VectorSubcoreMesh(**_SC_MESH),
        compiler_params=_SC_PARAMS,
        scratch_types=[
            pltpu.VMEM_SHARED((n_real, F), jnp.float32),
            pltpu.VMEM((ob_c, C), jnp.int32),
            pltpu.VMEM((ob_c, C), jnp.int32),
            pltpu.VMEM((C, F), jnp.float32),
            pltpu.VMEM((C, F), jnp.float32),
            pltpu.VMEM((C, F), jnp.bfloat16),
            pltpu.VMEM((C, F), jnp.bfloat16),
            pltpu.VMEM((C, F), jnp.float32),
            pltpu.VMEM((C, F), jnp.float32),
            pltpu.SemaphoreType.DMA,
            pltpu.SemaphoreType.DMA,
            pltpu.SemaphoreType.DMA,
            pltpu.SemaphoreType.DMA,
            pltpu.SemaphoreType.DMA,
            pltpu.SemaphoreType.DMA,
        ],
    )(functools.partial(_msg_body, n_real, n_chunks, rpt, zr, ob_n, ob_c))


def _embed_body(n_real, rb, z_ref, emb_ref, out_ref):
    i = pl.program_id(0)
    z = z_ref[...]
    oh = (z == lax.broadcasted_iota(jnp.int32, (rb, 128), 1)).astype(jnp.float32)
    x0 = jnp.dot(oh, emb_ref[...], preferred_element_type=jnp.float32)
    rows = i * rb + lax.broadcasted_iota(jnp.int32, (rb, 1), 0)
    out_ref[...] = jnp.where(rows < n_real, x0, 0.0)


def _g_body(e_real, gr, rsq_ref, wrad_ref, g0_ref, g1_ref, rbf_scr):
    i = pl.program_id(0)
    rsq = rsq_ref[...]
    r = jnp.sqrt(rsq + 1e-12)
    u = 2.0 * jnp.exp(-r) - 1.0
    x2 = (r / CUTOFF) ** 2
    cut = jnp.where(x2 < 1.0,
                    jnp.exp(1.0 - 1.0 / jnp.maximum(1.0 - x2, 1e-9)), 0.0)
    eid = (i * gr * 128
           + lax.broadcasted_iota(jnp.int32, (gr, 128), 0) * 128
           + lax.broadcasted_iota(jnp.int32, (gr, 128), 1))
    cut = jnp.where(eid < e_real, cut, 0.0)
    t_prev = cut
    t_cur = u * cut
    rbf_scr[0] = t_prev
    rbf_scr[1] = t_cur
    for k in range(2, NB):
        t_prev, t_cur = t_cur, 2.0 * u * t_cur - t_prev
        rbf_scr[k] = t_cur
    rbf_t = rbf_scr[...].reshape(NB, gr * 128)
    w = wrad_ref[...]
    g = lax.dot_general(rbf_t, w, (((0,), (0,)), ((), ())),
                        preferred_element_type=jnp.float32)
    g0_ref[...] = g[:, :F].astype(jnp.bfloat16).reshape(gr, 128, F)
    g1_ref[...] = g[:, F:].astype(jnp.bfloat16).reshape(gr, 128, F)


def _upd_body(n_real, rb, x_ref, agg_ref, w_ref, out_ref):
    i = pl.program_id(0)
    x = x_ref[...]
    rows = i * rb + lax.broadcasted_iota(jnp.int32, (rb, 1), 0)
    a = jnp.where(rows < n_real, agg_ref[0] + agg_ref[1], 0.0)
    w = w_ref[...]
    wm, w1, w2 = w[:F], w[F:2 * F], w[2 * F:]
    y = x + jnp.dot(a, wm, preferred_element_type=jnp.float32)
    for _ in range(NRES):
        h = jax.nn.relu(jnp.dot(y, w1, preferred_element_type=jnp.float32))
        y = y + jnp.dot(h, w2, preferred_element_type=jnp.float32)
    out_ref[...] = y


def _readout_body(nseg, rb, x_ref, am_ref, seg_ref, wout_ref, bm_ref, out_ref):
    i = pl.program_id(0)
    e = jnp.dot(x_ref[...], wout_ref[...],
                preferred_element_type=jnp.float32) * am_ref[...]
    oh = (seg_ref[...] == lax.broadcasted_iota(jnp.int32, (rb, nseg), 1)
          ).astype(jnp.float32)
    part = lax.dot_general(e, oh, (((0,), (0,)), ((), ())),
                           preferred_element_type=jnp.float32)

    @pl.when(i == 0)
    def _init():
        out_ref[...] = jnp.zeros_like(out_ref)

    out_ref[...] += part

    @pl.when(i == pl.num_programs(0) - 1)
    def _mask():
        out_ref[...] *= bm_ref[...]


def kernel(atomic_numbers, positions, dst_idx, src_idx, batch_segments,
           batch_size, batch_mask, atom_mask, embed, W_rad, W_msg, Wr1, Wr2,
           W_out):
    n = positions.shape[0]
    e = dst_idx.shape[0]
    nseg = batch_mask.shape[0]
    zmax = embed.shape[0]

    rb = 1024
    n_pad = -(-n // rb) * rb
    n_chunks = -(-e // (NW * C))
    e_pad = NW * n_chunks * C

    pos16 = jnp.pad(positions, ((0, n_pad - n), (0, 13)))
    srcr = jnp.pad(src_idx, (0, e_pad - e)).reshape(NW, n_chunks, C)
    dstr = jnp.pad(dst_idx, (0, e_pad - e)).reshape(NW, n_chunks, C)
    z_col = jnp.pad(atomic_numbers, (0, n_pad - n)).reshape(n_pad, 1)
    emb_pad = jnp.pad(embed, ((0, 128 - zmax), (0, 0)))
    am_col = jnp.pad(atom_mask, (0, n_pad - n)).reshape(n_pad, 1)
    seg_col = jnp.pad(batch_segments, (0, n_pad - n)).reshape(n_pad, 1)
    half = F // 2
    perm = jnp.stack([jnp.arange(half), half + jnp.arange(half)],
                     axis=1).reshape(F)
    wrad2 = jnp.concatenate([W_rad[0][:, perm], W_rad[1][:, perm]], axis=1)
    bm_row = batch_mask.reshape(1, nseg)

    rsq = _make_geom(n_pad, n_chunks)(pos16, srcr, dstr)

    gr = 32
    n_eblk = NW * n_chunks // gr
    g0, g1 = pl.pallas_call(
        functools.partial(_g_body, e, gr),
        grid=(n_eblk,),
        in_specs=[
            pl.BlockSpec((gr, C), lambda i: (i, 0)),
            pl.BlockSpec((NB, NIT * F), lambda i: (0, 0)),
        ],
        out_specs=[
            pl.BlockSpec((gr, C, F), lambda i: (i, 0, 0)),
            pl.BlockSpec((gr, C, F), lambda i: (i, 0, 0)),
        ],
        out_shape=[
            jax.ShapeDtypeStruct((NW * n_chunks, C, F), jnp.bfloat16),
            jax.ShapeDtypeStruct((NW * n_chunks, C, F), jnp.bfloat16),
        ],
        scratch_shapes=[pltpu.VMEM((NB, gr, C), jnp.float32)],
    )(rsq, wrad2)

    n_rblk = n_pad // rb
    x = pl.pallas_call(
        functools.partial(_embed_body, n, rb),
        grid=(n_rblk,),
        in_specs=[
            pl.BlockSpec((rb, 1), lambda i: (i, 0)),
            pl.BlockSpec((128, F), lambda i: (0, 0)),
        ],
        out_specs=pl.BlockSpec((rb, F), lambda i: (i, 0)),
        out_shape=jax.ShapeDtypeStruct((n_pad, F), jnp.float32),
    )(z_col, emb_pad)

    msg_call = _make_msg(n, n_chunks)
    ob_c = 14
    srcr3 = srcr.reshape(NW * (n_chunks // ob_c), ob_c, C)
    dstr3 = dstr.reshape(NW * (n_chunks // ob_c), ob_c, C)
    for i in range(NIT):
        g = (g0, g1)[i]
        agg = msg_call(x, g, srcr3, dstr3).reshape(NC, n, F)
        w_pack = jnp.concatenate([W_msg[i], Wr1[i], Wr2[i]], axis=0)
        x = pl.pallas_call(
            functools.partial(_upd_body, n, rb),
            grid=(n_rblk,),
            in_specs=[
                pl.BlockSpec((rb, F), lambda i: (i, 0)),
                pl.BlockSpec((NC, rb, F), lambda i: (0, i, 0)),
                pl.BlockSpec((3 * F, F), lambda i: (0, 0)),
            ],
            out_specs=pl.BlockSpec((rb, F), lambda i: (i, 0)),
            out_shape=jax.ShapeDtypeStruct((n_pad, F), jnp.float32),
        )(x, agg, w_pack)

    energy = pl.pallas_call(
        functools.partial(_readout_body, nseg, rb),
        grid=(n_rblk,),
        in_specs=[
            pl.BlockSpec((rb, F), lambda i: (i, 0)),
            pl.BlockSpec((rb, 1), lambda i: (i, 0)),
            pl.BlockSpec((rb, 1), lambda i: (i, 0)),
            pl.BlockSpec((F, 1), lambda i: (0, 0)),
            pl.BlockSpec((1, nseg), lambda i: (0, 0)),
        ],
        out_specs=pl.BlockSpec((1, nseg), lambda i: (0, 0)),
        out_shape=jax.ShapeDtypeStruct((1, nseg), jnp.float32),
    )(x, am_col, seg_col, W_out, bm_row)

    return energy[0]


# speedup vs baseline: 1.0776x; 1.0776x over previous
import functools

import jax
import jax.numpy as jnp
from jax import lax
from jax.experimental import pallas as pl
from jax.experimental.pallas import tpu as pltpu
from jax.experimental.pallas import tpu_sc as plsc

F = 32
NB = 16
NIT = 2
NRES = 3
CUTOFF = 6.0

NC = 2
NS = 16
NW = NC * NS
C = 128

_SC_MESH = dict(core_axis_name="c", subcore_axis_name="s",
                num_cores=NC, num_subcores=NS)
_SC_PARAMS = pltpu.CompilerParams(use_tc_tiling_on_sc=False)


def _vgather(x, idx):
    return lax.gather(
        x, idx[:, None],
        lax.GatherDimensionNumbers(offset_dims=(), collapsed_slice_dims=(0,),
                                   start_index_map=(0,)),
        (1,), mode=lax.GatherScatterMode.PROMISE_IN_BOUNDS)


def _geom_body(n_chunks, pos_hbm, srcr_hbm, dstr_hbm, rsq_hbm,
               src_v, dst_v, ps_v, pd_v, rs_v, sem1, sem2):
    cid = lax.axis_index("c")
    sid = lax.axis_index("s")
    wid = cid * NS + sid
    pltpu.sync_copy(srcr_hbm.at[wid], src_v)
    pltpu.sync_copy(dstr_hbm.at[wid], dst_v)
    lanes = lax.iota(jnp.int32, 16)
    ix = jnp.zeros((16,), jnp.int32)
    iy = jnp.full((16,), 1, jnp.int32)
    iz = jnp.full((16,), 2, jnp.int32)

    @pl.loop(0, n_chunks)
    def _chunk(j):
        cp1 = pltpu.async_copy(pos_hbm.at[src_v.at[j]], ps_v, sem1)
        cp2 = pltpu.async_copy(pos_hbm.at[dst_v.at[j]], pd_v, sem2)
        cp1.wait()
        cp2.wait()

        @pl.loop(0, C // 16)
        def _grp(gq):
            acc = jnp.zeros((16,), jnp.float32)
            for sub in range(16):
                r = gq * 16 + sub
                dv = ps_v[r, :] - pd_v[r, :]
                d2 = dv * dv
                rs = _vgather(d2, ix) + _vgather(d2, iy) + _vgather(d2, iz)
                acc = jnp.where(lanes == sub, rs, acc)
            rs_v[pl.ds(gq * 16, 16)] = acc

        pltpu.sync_copy(rs_v, rsq_hbm.at[wid * n_chunks + j])


def _make_geom(n_pad, n_chunks):
    return functools.partial(
        pl.kernel,
        out_type=jax.ShapeDtypeStruct((NW * n_chunks, C), jnp.float32),
        mesh=plsc.VectorSubcoreMesh(**_SC_MESH),
        compiler_params=_SC_PARAMS,
        scratch_types=[
            pltpu.VMEM((n_chunks, C), jnp.int32),
            pltpu.VMEM((n_chunks, C), jnp.int32),
            pltpu.VMEM((C, 16), jnp.float32),
            pltpu.VMEM((C, 16), jnp.float32),
            pltpu.VMEM((C,), jnp.float32),
            pltpu.SemaphoreType.DMA,
            pltpu.SemaphoreType.DMA,
        ],
    )(functools.partial(_geom_body, n_chunks))


def _msg_body(n_real, n_chunks, rpt, zr, ob_n, ob_c,
              x_hbm, g_hbm, srcr_hbm, dstr_hbm, agg_hbm,
              acc, si_s, di_s, xv0, xv1, gv0, gv1, mv0, mv1,
              sx0, sx1, sg0, sg1, ss0, ss1):
    cid = lax.axis_index("c")
    sid = lax.axis_index("s")
    wid = cid * NS + sid
    zero16 = jnp.zeros((16,), jnp.float32)

    @pl.loop(0, C)
    def _zrow(r):
        mv0[r, pl.ds(0, 16)] = zero16
        mv0[r, pl.ds(16, 16)] = zero16

    @pl.loop(0, rpt // zr)
    def _zacc(k):
        pltpu.sync_copy(mv0.at[pl.ds(0, zr)],
                        acc.at[pl.ds(sid * rpt + k * zr, zr)])

    plsc.subcore_barrier()

    def mult(xv, gv, mv):
        @pl.loop(0, C)
        def _mul(r):
            mv[r, pl.ds(0, 16)] = xv[r, pl.ds(0, 16)] * gv[r, pl.ds(0, 16)]
            mv[r, pl.ds(16, 16)] = xv[r, pl.ds(16, 16)] * gv[r, pl.ds(16, 16)]

    def drain_scatter(sem, mv):
        pltpu.make_async_copy(x_hbm.at[pl.ds(0, C)], mv, sem).wait()

    @pl.loop(0, ob_n)
    def _ob(ob):
        base = wid * n_chunks + ob * ob_c

        @pl.when(ob > 0)
        def _drain_prev():
            drain_scatter(ss0, mv0)
            drain_scatter(ss1, mv1)

        pltpu.sync_copy(srcr_hbm.at[(wid * ob_n + ob)], si_s)
        pltpu.sync_copy(dstr_hbm.at[(wid * ob_n + ob)], di_s)

        def fire(j, xv, gv, sx, sg):
            pltpu.async_copy(x_hbm.at[si_s.at[j]], xv, sx)
            pltpu.async_copy(g_hbm.at[base + j], gv, sg)

        def wait_loads(xv, gv, sx, sg):
            pltpu.make_async_copy(x_hbm.at[pl.ds(0, C)], xv, sx).wait()
            pltpu.make_async_copy(g_hbm.at[0], gv, sg).wait()

        fire(0, xv0, gv0, sx0, sg0)
        fire(1, xv1, gv1, sx1, sg1)

        @pl.loop(0, ob_c // 2)
        def _pair(p):
            j0 = 2 * p
            wait_loads(xv0, gv0, sx0, sg0)
            mult(xv0, gv0, mv0)

            @pl.when(p < ob_c // 2 - 1)
            def _pf0():
                fire(j0 + 2, xv0, gv0, sx0, sg0)

            @pl.when(p > 0)
            def _dr0():
                drain_scatter(ss0, mv0)

            pltpu.async_copy(mv0, acc.at[di_s.at[j0]], ss0, add=True)

            wait_loads(xv1, gv1, sx1, sg1)
            mult(xv1, gv1, mv1)

            @pl.when(p < ob_c // 2 - 1)
            def _pf1():
                fire(j0 + 3, xv1, gv1, sx1, sg1)

            @pl.when(p > 0)
            def _dr1():
                drain_scatter(ss1, mv1)

            pltpu.async_copy(mv1, acc.at[di_s.at[j0 + 1]], ss1, add=True)

    drain_scatter(ss0, mv0)
    drain_scatter(ss1, mv1)
    plsc.subcore_barrier()
    pltpu.sync_copy(acc.at[pl.ds(sid * rpt, rpt)],
                    agg_hbm.at[pl.ds(cid * n_real + sid * rpt, rpt)])


def _make_msg(n_real, n_chunks):
    rpt = n_real // NS
    zr = 125
    ob_c = 14
    ob_n = n_chunks // ob_c
    return functools.partial(
        pl.kernel,
        out_type=jax.ShapeDtypeStruct((NC * n_real, F), jnp.float32),
        mesh=plsc.VectorSubcoreMesh(**_SC_MESH),
        compiler_params=_SC_PARAMS,
        scratch_types=[
            pltpu.VMEM_SHARED((n_real, F), jnp.float32),
            pltpu.VMEM((ob_c, C), jnp.int32),
            pltpu.VMEM((ob_c, C), jnp.int32),
            pltpu.VMEM((C, F), jnp.float32),
            pltpu.VMEM((C, F), jnp.float32),
            pltpu.VMEM((C, F), jnp.float32),
            pltpu.VMEM((C, F), jnp.float32),
            pltpu.VMEM((C, F), jnp.float32),
            pltpu.VMEM((C, F), jnp.float32),
            pltpu.SemaphoreType.DMA,
            pltpu.SemaphoreType.DMA,
            pltpu.SemaphoreType.DMA,
            pltpu.SemaphoreType.DMA,
            pltpu.SemaphoreType.DMA,
            pltpu.SemaphoreType.DMA,
        ],
    )(functools.partial(_msg_body, n_real, n_chunks, rpt, zr, ob_n, ob_c))


def _embed_body(n_real, rb, z_ref, emb_ref, out_ref):
    i = pl.program_id(0)
    z = z_ref[...]
    oh = (z == lax.broadcasted_iota(jnp.int32, (rb, 128), 1)).astype(jnp.float32)
    x0 = jnp.dot(oh, emb_ref[...], preferred_element_type=jnp.float32)
    rows = i * rb + lax.broadcasted_iota(jnp.int32, (rb, 1), 0)
    out_ref[...] = jnp.where(rows < n_real, x0, 0.0)


def _g_body(e_real, gr, rsq_ref, wrad_ref, g0_ref, g1_ref, rbf_scr):
    i = pl.program_id(0)
    rsq = rsq_ref[...]
    r = jnp.sqrt(rsq + 1e-12)
    u = 2.0 * jnp.exp(-r) - 1.0
    x2 = (r / CUTOFF) ** 2
    cut = jnp.where(x2 < 1.0,
                    jnp.exp(1.0 - 1.0 / jnp.maximum(1.0 - x2, 1e-9)), 0.0)
    eid = (i * gr * 128
           + lax.broadcasted_iota(jnp.int32, (gr, 128), 0) * 128
           + lax.broadcasted_iota(jnp.int32, (gr, 128), 1))
    cut = jnp.where(eid < e_real, cut, 0.0)
    t_prev = cut
    t_cur = u * cut
    rbf_scr[0] = t_prev
    rbf_scr[1] = t_cur
    for k in range(2, NB):
        t_prev, t_cur = t_cur, 2.0 * u * t_cur - t_prev
        rbf_scr[k] = t_cur
    rbf_t = rbf_scr[...].reshape(NB, gr * 128)
    w = wrad_ref[...]
    g = lax.dot_general(rbf_t, w, (((0,), (0,)), ((), ())),
                        preferred_element_type=jnp.float32)
    g0_ref[...] = g[:, :F].reshape(gr, 128, F)
    g1_ref[...] = g[:, F:].reshape(gr, 128, F)


def _upd_body(n_real, rb, x_ref, agg_ref, w_ref, out_ref):
    i = pl.program_id(0)
    x = x_ref[...]
    rows = i * rb + lax.broadcasted_iota(jnp.int32, (rb, 1), 0)
    a = jnp.where(rows < n_real, agg_ref[0] + agg_ref[1], 0.0)
    w = w_ref[...]
    wm, w1, w2 = w[:F], w[F:2 * F], w[2 * F:]
    y = x + jnp.dot(a, wm, preferred_element_type=jnp.float32)
    for _ in range(NRES):
        h = jax.nn.relu(jnp.dot(y, w1, preferred_element_type=jnp.float32))
        y = y + jnp.dot(h, w2, preferred_element_type=jnp.float32)
    out_ref[...] = y


def _readout_body(nseg, rb, x_ref, am_ref, seg_ref, wout_ref, bm_ref, out_ref):
    i = pl.program_id(0)
    e = jnp.dot(x_ref[...], wout_ref[...],
                preferred_element_type=jnp.float32) * am_ref[...]
    oh = (seg_ref[...] == lax.broadcasted_iota(jnp.int32, (rb, nseg), 1)
          ).astype(jnp.float32)
    part = lax.dot_general(e, oh, (((0,), (0,)), ((), ())),
                           preferred_element_type=jnp.float32)

    @pl.when(i == 0)
    def _init():
        out_ref[...] = jnp.zeros_like(out_ref)

    out_ref[...] += part

    @pl.when(i == pl.num_programs(0) - 1)
    def _mask():
        out_ref[...] *= bm_ref[...]


def kernel(atomic_numbers, positions, dst_idx, src_idx, batch_segments,
           batch_size, batch_mask, atom_mask, embed, W_rad, W_msg, Wr1, Wr2,
           W_out):
    n = positions.shape[0]
    e = dst_idx.shape[0]
    nseg = batch_mask.shape[0]
    zmax = embed.shape[0]

    rb = 1024
    n_pad = -(-n // rb) * rb
    n_chunks = -(-e // (NW * C))
    e_pad = NW * n_chunks * C

    pos16 = jnp.pad(positions, ((0, n_pad - n), (0, 13)))
    srcr = jnp.pad(src_idx, (0, e_pad - e)).reshape(NW, n_chunks, C)
    dstr = jnp.pad(dst_idx, (0, e_pad - e)).reshape(NW, n_chunks, C)
    z_col = jnp.pad(atomic_numbers, (0, n_pad - n)).reshape(n_pad, 1)
    emb_pad = jnp.pad(embed, ((0, 128 - zmax), (0, 0)))
    am_col = jnp.pad(atom_mask, (0, n_pad - n)).reshape(n_pad, 1)
    seg_col = jnp.pad(batch_segments, (0, n_pad - n)).reshape(n_pad, 1)
    wrad2 = jnp.concatenate([W_rad[0], W_rad[1]], axis=1)
    bm_row = batch_mask.reshape(1, nseg)

    rsq = _make_geom(n_pad, n_chunks)(pos16, srcr, dstr)

    gr = 32
    n_eblk = NW * n_chunks // gr
    g0, g1 = pl.pallas_call(
        functools.partial(_g_body, e, gr),
        grid=(n_eblk,),
        in_specs=[
            pl.BlockSpec((gr, C), lambda i: (i, 0)),
            pl.BlockSpec((NB, NIT * F), lambda i: (0, 0)),
        ],
        out_specs=[
            pl.BlockSpec((gr, C, F), lambda i: (i, 0, 0)),
            pl.BlockSpec((gr, C, F), lambda i: (i, 0, 0)),
        ],
        out_shape=[
            jax.ShapeDtypeStruct((NW * n_chunks, C, F), jnp.float32),
            jax.ShapeDtypeStruct((NW * n_chunks, C, F), jnp.float32),
        ],
        scratch_shapes=[pltpu.VMEM((NB, gr, C), jnp.float32)],
    )(rsq, wrad2)

    n_rblk = n_pad // rb
    x = pl.pallas_call(
        functools.partial(_embed_body, n, rb),
        grid=(n_rblk,),
        in_specs=[
            pl.BlockSpec((rb, 1), lambda i: (i, 0)),
            pl.BlockSpec((128, F), lambda i: (0, 0)),
        ],
        out_specs=pl.BlockSpec((rb, F), lambda i: (i, 0)),
        out_shape=jax.ShapeDtypeStruct((n_pad, F), jnp.float32),
    )(z_col, emb_pad)

    msg_call = _make_msg(n, n_chunks)
    ob_c = 14
    srcr3 = srcr.reshape(NW * (n_chunks // ob_c), ob_c, C)
    dstr3 = dstr.reshape(NW * (n_chunks // ob_c), ob_c, C)
    for i in range(NIT):
        g = (g0, g1)[i]
        agg = msg_call(x, g, srcr3, dstr3).reshape(NC, n, F)
        w_pack = jnp.concatenate([W_msg[i], Wr1[i], Wr2[i]], axis=0)
        x = pl.pallas_call(
            functools.partial(_upd_body, n, rb),
            grid=(n_rblk,),
            in_specs=[
                pl.BlockSpec((rb, F), lambda i: (i, 0)),
                pl.BlockSpec((NC, rb, F), lambda i: (0, i, 0)),
                pl.BlockSpec((3 * F, F), lambda i: (0, 0)),
            ],
            out_specs=pl.BlockSpec((rb, F), lambda i: (i, 0)),
            out_shape=jax.ShapeDtypeStruct((n_pad, F), jnp.float32),
        )(x, agg, w_pack)

    energy = pl.pallas_call(
        functools.partial(_readout_body, nseg, rb),
        grid=(n_rblk,),
        in_specs=[
            pl.BlockSpec((rb, F), lambda i: (i, 0)),
            pl.BlockSpec((rb, 1), lambda i: (i, 0)),
            pl.BlockSpec((rb, 1), lambda i: (i, 0)),
            pl.BlockSpec((F, 1), lambda i: (0, 0)),
            pl.BlockSpec((1, nseg), lambda i: (0, 0)),
        ],
        out_specs=pl.BlockSpec((1, nseg), lambda i: (0, 0)),
        out_shape=jax.ShapeDtypeStruct((1, nseg), jnp.float32),
    )(x, am_col, seg_col, W_out, bm_row)

    return energy[0]


# speedup vs baseline: 1.1505x; 1.0677x over previous
import functools

import jax
import jax.numpy as jnp
from jax import lax
from jax.experimental import pallas as pl
from jax.experimental.pallas import tpu as pltpu
from jax.experimental.pallas import tpu_sc as plsc

F = 32
NB = 16
NIT = 2
NRES = 3
CUTOFF = 6.0

NC = 2
NS = 16
NW = NC * NS
C = 128

_SC_MESH = dict(core_axis_name="c", subcore_axis_name="s",
                num_cores=NC, num_subcores=NS)
_SC_PARAMS = pltpu.CompilerParams(use_tc_tiling_on_sc=False)


def _vgather(x, idx):
    return lax.gather(
        x, idx[:, None],
        lax.GatherDimensionNumbers(offset_dims=(), collapsed_slice_dims=(0,),
                                   start_index_map=(0,)),
        (1,), mode=lax.GatherScatterMode.PROMISE_IN_BOUNDS)


def _geom_body(n_chunks, pos_hbm, srcr_hbm, dstr_hbm, rsq_hbm,
               src_v, dst_v, psa, pda, psb, pdb, rsa, rsb,
               sa1, sa2, sb1, sb2):
    cid = lax.axis_index("c")
    sid = lax.axis_index("s")
    wid = cid * NS + sid
    pltpu.sync_copy(srcr_hbm.at[wid], src_v)
    pltpu.sync_copy(dstr_hbm.at[wid], dst_v)
    lanes = lax.iota(jnp.int32, 16)
    ix = jnp.zeros((16,), jnp.int32)
    iy = jnp.full((16,), 1, jnp.int32)
    iz = jnp.full((16,), 2, jnp.int32)

    def fire(j, ps, pd, s1, s2):
        pltpu.async_copy(pos_hbm.at[src_v.at[j]], ps, s1)
        pltpu.async_copy(pos_hbm.at[dst_v.at[j]], pd, s2)

    def wait_pair(ps, pd, s1, s2):
        pltpu.make_async_copy(pos_hbm.at[pl.ds(0, C)], ps, s1).wait()
        pltpu.make_async_copy(pos_hbm.at[pl.ds(0, C)], pd, s2).wait()

    def compute(ps, pd, rs):
        @pl.loop(0, C // 16)
        def _grp(gq):
            acc = jnp.zeros((16,), jnp.float32)
            for sub in range(16):
                r = gq * 16 + sub
                dv = ps[r, :] - pd[r, :]
                d2 = dv * dv
                v = _vgather(d2, ix) + _vgather(d2, iy) + _vgather(d2, iz)
                acc = jnp.where(lanes == sub, v, acc)
            rs[pl.ds(gq * 16, 16)] = acc

    fire(0, psa, pda, sa1, sa2)
    fire(1, psb, pdb, sb1, sb2)

    @pl.loop(0, n_chunks // 2)
    def _pair(p):
        j0 = 2 * p
        wait_pair(psa, pda, sa1, sa2)
        compute(psa, pda, rsa)

        @pl.when(p < n_chunks // 2 - 1)
        def _pfa():
            fire(j0 + 2, psa, pda, sa1, sa2)

        pltpu.sync_copy(rsa, rsq_hbm.at[wid * n_chunks + j0])

        wait_pair(psb, pdb, sb1, sb2)
        compute(psb, pdb, rsb)

        @pl.when(p < n_chunks // 2 - 1)
        def _pfb():
            fire(j0 + 3, psb, pdb, sb1, sb2)

        pltpu.sync_copy(rsb, rsq_hbm.at[wid * n_chunks + j0 + 1])


def _make_geom(n_pad, n_chunks):
    return functools.partial(
        pl.kernel,
        out_type=jax.ShapeDtypeStruct((NW * n_chunks, C), jnp.float32),
        mesh=plsc.VectorSubcoreMesh(**_SC_MESH),
        compiler_params=_SC_PARAMS,
        scratch_types=[
            pltpu.VMEM((n_chunks, C), jnp.int32),
            pltpu.VMEM((n_chunks, C), jnp.int32),
            pltpu.VMEM((C, 16), jnp.float32),
            pltpu.VMEM((C, 16), jnp.float32),
            pltpu.VMEM((C, 16), jnp.float32),
            pltpu.VMEM((C, 16), jnp.float32),
            pltpu.VMEM((C,), jnp.float32),
            pltpu.VMEM((C,), jnp.float32),
            pltpu.SemaphoreType.DMA,
            pltpu.SemaphoreType.DMA,
            pltpu.SemaphoreType.DMA,
            pltpu.SemaphoreType.DMA,
        ],
    )(functools.partial(_geom_body, n_chunks))


def _msg_body(n_real, n_chunks, rpt, zr, ob_n, ob_c,
              x_hbm, g_hbm, srcr_hbm, dstr_hbm, agg_hbm,
              acc, si_s, di_s, xv0, xv1, gv0, gv1, mv0, mv1,
              sx0, sx1, sg0, sg1, ss0, ss1):
    cid = lax.axis_index("c")
    sid = lax.axis_index("s")
    wid = cid * NS + sid
    zero16 = jnp.zeros((16,), jnp.float32)

    @pl.loop(0, C)
    def _zrow(r):
        mv0[r, pl.ds(0, 16)] = zero16
        mv0[r, pl.ds(16, 16)] = zero16

    @pl.loop(0, rpt // zr)
    def _zacc(k):
        pltpu.sync_copy(mv0.at[pl.ds(0, zr)],
                        acc.at[pl.ds(sid * rpt + k * zr, zr)])

    plsc.subcore_barrier()

    def mult(xv, gv, mv):
        @pl.loop(0, C)
        def _mul(r):
            mv[r, pl.ds(0, 16)] = xv[r, pl.ds(0, 16)] * gv[r, pl.ds(0, 16)]
            mv[r, pl.ds(16, 16)] = xv[r, pl.ds(16, 16)] * gv[r, pl.ds(16, 16)]

    def drain_scatter(sem, mv):
        pltpu.make_async_copy(x_hbm.at[pl.ds(0, C)], mv, sem).wait()

    @pl.loop(0, ob_n)
    def _ob(ob):
        base = wid * n_chunks + ob * ob_c

        @pl.when(ob > 0)
        def _drain_prev():
            drain_scatter(ss0, mv0)
            drain_scatter(ss1, mv1)

        pltpu.sync_copy(srcr_hbm.at[(wid * ob_n + ob)], si_s)
        pltpu.sync_copy(dstr_hbm.at[(wid * ob_n + ob)], di_s)

        def fire(j, xv, gv, sx, sg):
            pltpu.async_copy(x_hbm.at[si_s.at[j]], xv, sx)
            pltpu.async_copy(g_hbm.at[base + j], gv, sg)

        def wait_loads(xv, gv, sx, sg):
            pltpu.make_async_copy(x_hbm.at[pl.ds(0, C)], xv, sx).wait()
            pltpu.make_async_copy(g_hbm.at[0], gv, sg).wait()

        fire(0, xv0, gv0, sx0, sg0)
        fire(1, xv1, gv1, sx1, sg1)

        @pl.loop(0, ob_c // 2)
        def _pair(p):
            j0 = 2 * p
            wait_loads(xv0, gv0, sx0, sg0)
            mult(xv0, gv0, mv0)

            @pl.when(p < ob_c // 2 - 1)
            def _pf0():
                fire(j0 + 2, xv0, gv0, sx0, sg0)

            @pl.when(p > 0)
            def _dr0():
                drain_scatter(ss0, mv0)

            pltpu.async_copy(mv0, acc.at[di_s.at[j0]], ss0, add=True)

            wait_loads(xv1, gv1, sx1, sg1)
            mult(xv1, gv1, mv1)

            @pl.when(p < ob_c // 2 - 1)
            def _pf1():
                fire(j0 + 3, xv1, gv1, sx1, sg1)

            @pl.when(p > 0)
            def _dr1():
                drain_scatter(ss1, mv1)

            pltpu.async_copy(mv1, acc.at[di_s.at[j0 + 1]], ss1, add=True)

    drain_scatter(ss0, mv0)
    drain_scatter(ss1, mv1)
    plsc.subcore_barrier()
    pltpu.sync_copy(acc.at[pl.ds(sid * rpt, rpt)],
                    agg_hbm.at[pl.ds(cid * n_real + sid * rpt, rpt)])


def _make_msg(n_real, n_chunks):
    rpt = n_real // NS
    zr = 125
    ob_c = 14
    ob_n = n_chunks // ob_c
    return functools.partial(
        pl.kernel,
        out_type=jax.ShapeDtypeStruct((NC * n_real, F), jnp.float32),
        mesh=plsc.VectorSubcoreMesh(**_SC_MESH),
        compiler_params=_SC_PARAMS,
        scratch_types=[
            pltpu.VMEM_SHARED((n_real, F), jnp.float32),
            pltpu.VMEM((ob_c, C), jnp.int32),
            pltpu.VMEM((ob_c, C), jnp.int32),
            pltpu.VMEM((C, F), jnp.float32),
            pltpu.VMEM((C, F), jnp.float32),
            pltpu.VMEM((C, F), jnp.float32),
            pltpu.VMEM((C, F), jnp.float32),
            pltpu.VMEM((C, F), jnp.float32),
            pltpu.VMEM((C, F), jnp.float32),
            pltpu.SemaphoreType.DMA,
            pltpu.SemaphoreType.DMA,
            pltpu.SemaphoreType.DMA,
            pltpu.SemaphoreType.DMA,
            pltpu.SemaphoreType.DMA,
            pltpu.SemaphoreType.DMA,
        ],
    )(functools.partial(_msg_body, n_real, n_chunks, rpt, zr, ob_n, ob_c))


def _embed_body(n_real, rb, z_ref, emb_ref, out_ref):
    i = pl.program_id(0)
    z = z_ref[...]
    oh = (z == lax.broadcasted_iota(jnp.int32, (rb, 128), 1)).astype(jnp.float32)
    x0 = jnp.dot(oh, emb_ref[...], preferred_element_type=jnp.float32)
    rows = i * rb + lax.broadcasted_iota(jnp.int32, (rb, 1), 0)
    out_ref[...] = jnp.where(rows < n_real, x0, 0.0)


def _g_body(e_real, gr, rsq_ref, wrad_ref, g0_ref, g1_ref, rbf_scr):
    i = pl.program_id(0)
    rsq = rsq_ref[...]
    r = jnp.sqrt(rsq + 1e-12)
    u = 2.0 * jnp.exp(-r) - 1.0
    x2 = (r / CUTOFF) ** 2
    cut = jnp.where(x2 < 1.0,
                    jnp.exp(1.0 - 1.0 / jnp.maximum(1.0 - x2, 1e-9)), 0.0)
    eid = (i * gr * 128
           + lax.broadcasted_iota(jnp.int32, (gr, 128), 0) * 128
           + lax.broadcasted_iota(jnp.int32, (gr, 128), 1))
    cut = jnp.where(eid < e_real, cut, 0.0)
    t_prev = cut
    t_cur = u * cut
    rbf_scr[0] = t_prev
    rbf_scr[1] = t_cur
    for k in range(2, NB):
        t_prev, t_cur = t_cur, 2.0 * u * t_cur - t_prev
        rbf_scr[k] = t_cur
    rbf_t = rbf_scr[...].reshape(NB, gr * 128)
    w = wrad_ref[...]
    g = lax.dot_general(rbf_t, w, (((0,), (0,)), ((), ())),
                        preferred_element_type=jnp.float32)
    g0_ref[...] = g[:, :F].reshape(gr, 128, F)
    g1_ref[...] = g[:, F:].reshape(gr, 128, F)


def _upd_body(n_real, rb, x_ref, agg_ref, w_ref, out_ref):
    i = pl.program_id(0)
    x = x_ref[...]
    rows = i * rb + lax.broadcasted_iota(jnp.int32, (rb, 1), 0)
    a = jnp.where(rows < n_real, agg_ref[0] + agg_ref[1], 0.0)
    w = w_ref[...]
    wm, w1, w2 = w[:F], w[F:2 * F], w[2 * F:]
    y = x + jnp.dot(a, wm, preferred_element_type=jnp.float32)
    for _ in range(NRES):
        h = jax.nn.relu(jnp.dot(y, w1, preferred_element_type=jnp.float32))
        y = y + jnp.dot(h, w2, preferred_element_type=jnp.float32)
    out_ref[...] = y


def _readout_body(nseg, rb, x_ref, am_ref, seg_ref, wout_ref, bm_ref, out_ref):
    i = pl.program_id(0)
    e = jnp.dot(x_ref[...], wout_ref[...],
                preferred_element_type=jnp.float32) * am_ref[...]
    oh = (seg_ref[...] == lax.broadcasted_iota(jnp.int32, (rb, nseg), 1)
          ).astype(jnp.float32)
    part = lax.dot_general(e, oh, (((0,), (0,)), ((), ())),
                           preferred_element_type=jnp.float32)

    @pl.when(i == 0)
    def _init():
        out_ref[...] = jnp.zeros_like(out_ref)

    out_ref[...] += part

    @pl.when(i == pl.num_programs(0) - 1)
    def _mask():
        out_ref[...] *= bm_ref[...]


def kernel(atomic_numbers, positions, dst_idx, src_idx, batch_segments,
           batch_size, batch_mask, atom_mask, embed, W_rad, W_msg, Wr1, Wr2,
           W_out):
    n = positions.shape[0]
    e = dst_idx.shape[0]
    nseg = batch_mask.shape[0]
    zmax = embed.shape[0]

    rb = 1024
    n_pad = -(-n // rb) * rb
    n_chunks = -(-e // (NW * C))
    e_pad = NW * n_chunks * C

    pos16 = jnp.pad(positions, ((0, n_pad - n), (0, 13)))
    srcr = jnp.pad(src_idx, (0, e_pad - e)).reshape(NW, n_chunks, C)
    dstr = jnp.pad(dst_idx, (0, e_pad - e)).reshape(NW, n_chunks, C)
    z_col = jnp.pad(atomic_numbers, (0, n_pad - n)).reshape(n_pad, 1)
    emb_pad = jnp.pad(embed, ((0, 128 - zmax), (0, 0)))
    am_col = jnp.pad(atom_mask, (0, n_pad - n)).reshape(n_pad, 1)
    seg_col = jnp.pad(batch_segments, (0, n_pad - n)).reshape(n_pad, 1)
    wrad2 = jnp.concatenate([W_rad[0], W_rad[1]], axis=1)
    bm_row = batch_mask.reshape(1, nseg)

    rsq = _make_geom(n_pad, n_chunks)(pos16, srcr, dstr)

    gr = 32
    n_eblk = NW * n_chunks // gr
    g0, g1 = pl.pallas_call(
        functools.partial(_g_body, e, gr),
        grid=(n_eblk,),
        in_specs=[
            pl.BlockSpec((gr, C), lambda i: (i, 0)),
            pl.BlockSpec((NB, NIT * F), lambda i: (0, 0)),
        ],
        out_specs=[
            pl.BlockSpec((gr, C, F), lambda i: (i, 0, 0)),
            pl.BlockSpec((gr, C, F), lambda i: (i, 0, 0)),
        ],
        out_shape=[
            jax.ShapeDtypeStruct((NW * n_chunks, C, F), jnp.float32),
            jax.ShapeDtypeStruct((NW * n_chunks, C, F), jnp.float32),
        ],
        scratch_shapes=[pltpu.VMEM((NB, gr, C), jnp.float32)],
    )(rsq, wrad2)

    n_rblk = n_pad // rb
    x = pl.pallas_call(
        functools.partial(_embed_body, n, rb),
        grid=(n_rblk,),
        in_specs=[
            pl.BlockSpec((rb, 1), lambda i: (i, 0)),
            pl.BlockSpec((128, F), lambda i: (0, 0)),
        ],
        out_specs=pl.BlockSpec((rb, F), lambda i: (i, 0)),
        out_shape=jax.ShapeDtypeStruct((n_pad, F), jnp.float32),
    )(z_col, emb_pad)

    msg_call = _make_msg(n, n_chunks)
    ob_c = 14
    srcr3 = srcr.reshape(NW * (n_chunks // ob_c), ob_c, C)
    dstr3 = dstr.reshape(NW * (n_chunks // ob_c), ob_c, C)
    for i in range(NIT):
        g = (g0, g1)[i]
        agg = msg_call(x, g, srcr3, dstr3).reshape(NC, n, F)
        w_pack = jnp.concatenate([W_msg[i], Wr1[i], Wr2[i]], axis=0)
        x = pl.pallas_call(
            functools.partial(_upd_body, n, rb),
            grid=(n_rblk,),
            in_specs=[
                pl.BlockSpec((rb, F), lambda i: (i, 0)),
                pl.BlockSpec((NC, rb, F), lambda i: (0, i, 0)),
                pl.BlockSpec((3 * F, F), lambda i: (0, 0)),
            ],
            out_specs=pl.BlockSpec((rb, F), lambda i: (i, 0)),
            out_shape=jax.ShapeDtypeStruct((n_pad, F), jnp.float32),
        )(x, agg, w_pack)

    energy = pl.pallas_call(
        functools.partial(_readout_body, nseg, rb),
        grid=(n_rblk,),
        in_specs=[
            pl.BlockSpec((rb, F), lambda i: (i, 0)),
            pl.BlockSpec((rb, 1), lambda i: (i, 0)),
            pl.BlockSpec((rb, 1), lambda i: (i, 0)),
            pl.BlockSpec((F, 1), lambda i: (0, 0)),
            pl.BlockSpec((1, nseg), lambda i: (0, 0)),
        ],
        out_specs=pl.BlockSpec((1, nseg), lambda i: (0, 0)),
        out_shape=jax.ShapeDtypeStruct((1, nseg), jnp.float32),
    )(x, am_col, seg_col, W_out, bm_row)

    return energy[0]
